# baseline (speedup 1.0000x reference)
import functools

import jax
import jax.numpy as jnp
from jax import lax
from jax.experimental import pallas as pl
from jax.experimental.pallas import tpu as pltpu
from jax.experimental.pallas import tpu_sc as plsc

DM = 768
NE = 8
FF = 2048
SEQ = 2048
TILE_M = 256
NUM_TILES = SEQ // TILE_M
NUM_ITEMS = NUM_TILES * NE
VT = 2048


def _sc_worker_id():
    return lax.axis_index("s") * 2 + lax.axis_index("c")


def _make_sc_gather(n_rows):
    info = plsc.get_sparse_core_info()
    nw = info.num_cores * info.num_subcores
    b_per_w = n_rows // nw
    mesh = plsc.VectorSubcoreMesh(core_axis_name="c", subcore_axis_name="s")

    @functools.partial(
        pl.kernel, mesh=mesh,
        out_type=jax.ShapeDtypeStruct((n_rows, DM), jnp.float32),
        scratch_types=[
            pltpu.VMEM((b_per_w,), jnp.int32),
            pltpu.VMEM((b_per_w, DM), jnp.float32),
            pltpu.SemaphoreType.DMA,
        ],
    )
    def k(table_hbm, idx_hbm, out_hbm, idx_v, rows_v, sem):
        base = _sc_worker_id() * b_per_w
        pltpu.sync_copy(idx_hbm.at[pl.ds(base, b_per_w)], idx_v)
        pltpu.async_copy(table_hbm.at[idx_v], rows_v, sem).wait()
        pltpu.sync_copy(rows_v, out_hbm.at[pl.ds(base, b_per_w)])

    return k


def _make_sc_scatter(n_rows):
    info = plsc.get_sparse_core_info()
    nw = info.num_cores * info.num_subcores
    b_per_w = n_rows // nw
    mesh = plsc.VectorSubcoreMesh(core_axis_name="c", subcore_axis_name="s")

    @functools.partial(
        pl.kernel, mesh=mesh,
        out_type=jax.ShapeDtypeStruct((n_rows, DM), jnp.float32),
        scratch_types=[
            pltpu.VMEM((b_per_w,), jnp.int32),
            pltpu.VMEM((b_per_w, DM), jnp.float32),
            pltpu.SemaphoreType.DMA,
        ],
    )
    def k(rows_hbm, pos_hbm, out_hbm, idx_v, rows_v, sem):
        base = _sc_worker_id() * b_per_w
        pltpu.sync_copy(pos_hbm.at[pl.ds(base, b_per_w)], idx_v)
        pltpu.sync_copy(rows_hbm.at[pl.ds(base, b_per_w)], rows_v)
        pltpu.async_copy(rows_v, out_hbm.at[idx_v], sem).wait()

    return k


def _router_body(x_ref, mu_ref, var_ref, g_ref, b_ref, wr_ref, br_ref,
                 h_ref, ex_ref, idx_ref, pos_ref, starts_ref):
    x = x_ref[...]
    h = ((x - mu_ref[...]) / jnp.sqrt(var_ref[...] + 1e-5) * g_ref[...]
         + b_ref[...])
    h_ref[...] = h
    logits = jnp.dot(h.astype(jnp.bfloat16),
                     wr_ref[...].astype(jnp.bfloat16),
                     preferred_element_type=jnp.float32)
    logits = logits + br_ref[...]
    mx = jnp.max(logits, axis=-1, keepdims=True)
    ex = jnp.exp(logits - mx)
    ex_ref[...] = ex
    idx = jnp.argmax(ex, axis=-1)
    idx_ref[...] = idx[:, None].astype(jnp.int32)
    onehot = (idx[:, None] == lax.broadcasted_iota(jnp.int32, (SEQ, NE), 1))
    onehot = onehot.astype(jnp.int32)
    csum = onehot
    k = 1
    while k < SEQ:
        shifted = jnp.concatenate(
            [jnp.zeros((k, NE), csum.dtype), csum[:SEQ - k]], axis=0)
        csum = csum + shifted
        k *= 2
    ranks = csum - onehot
    counts = csum[SEQ - 1:SEQ, :]
    er = lax.broadcasted_iota(jnp.int32, (NE, NE), 0)
    ec = lax.broadcasted_iota(jnp.int32, (NE, NE), 1)
    cmat = jnp.where(er < ec, jnp.broadcast_to(counts.reshape(NE, 1),
                                               (NE, NE)), 0)
    starts = jnp.sum(cmat, axis=0, keepdims=True)
    pos = jnp.sum(onehot * (starts + ranks), axis=-1, keepdims=True)
    pos_ref[...] = pos
    starts_ref[...] = starts


def _router_call(x, mu, var, g, b, wr, br):
    f32 = jnp.float32
    outs = (
        jax.ShapeDtypeStruct((SEQ, DM), f32),
        jax.ShapeDtypeStruct((SEQ, NE), f32),
        jax.ShapeDtypeStruct((SEQ, 1), jnp.int32),
        jax.ShapeDtypeStruct((SEQ, 1), jnp.int32),
        jax.ShapeDtypeStruct((1, NE), jnp.int32),
    )
    return pl.pallas_call(_router_body, out_shape=outs)(
        x, mu, var, g, b, wr, br)


def _final_ln_body(x_ref, mu_ref, var_ref, g_ref, b_ref, xf_ref):
    x = x_ref[...]
    xf_ref[...] = ((x - mu_ref[...]) / jnp.sqrt(var_ref[...] + 1e-5)
                   * g_ref[...] + b_ref[...])


def _final_ln_call(x, mu, var, g, b):
    return pl.pallas_call(
        _final_ln_body,
        out_shape=jax.ShapeDtypeStruct((SEQ, DM), jnp.float32),
    )(x, mu, var, g, b)


def _ffn1_body(e_ref, lo_ref, hi_ref, xs_ref, w1_ref, b1_ref, hid_ref):
    j = pl.program_id(0)
    lo = lo_ref[j]
    hi = hi_ref[j]

    @pl.when(hi > lo)
    def _():
        xt = xs_ref[...].astype(jnp.bfloat16)
        a = jnp.dot(xt, w1_ref[0].astype(jnp.bfloat16),
                    preferred_element_type=jnp.float32)
        a = a + b1_ref[0]
        hmid = jax.nn.gelu(a)
        rows = lax.broadcasted_iota(jnp.int32, (TILE_M, 1), 0)
        mask = (rows >= lo) & (rows < hi)
        hid_ref[...] = jnp.where(mask, hmid, hid_ref[...])


def _ffn2_body(e_ref, lo_ref, hi_ref, hid_ref, w2_ref, b2_ref, ys_ref):
    j = pl.program_id(0)
    lo = lo_ref[j]
    hi = hi_ref[j]

    @pl.when(hi > lo)
    def _():
        y = jnp.dot(hid_ref[...].astype(jnp.bfloat16),
                    w2_ref[0].astype(jnp.bfloat16),
                    preferred_element_type=jnp.float32)
        y = y + b2_ref[0]
        rows = lax.broadcasted_iota(jnp.int32, (TILE_M, 1), 0)
        mask = (rows >= lo) & (rows < hi)
        ys_ref[...] = jnp.where(mask, y, ys_ref[...])


def _ffn_call(e_eff, lo, hi, xs, w1, b1, w2, b2):
    spec1 = pltpu.PrefetchScalarGridSpec(
        num_scalar_prefetch=3,
        grid=(NUM_ITEMS,),
        in_specs=[
            pl.BlockSpec((TILE_M, DM), lambda j, e, lo, hi: (j // NE, 0)),
            pl.BlockSpec((1, DM, FF), lambda j, e, lo, hi: (e[j], 0, 0)),
            pl.BlockSpec((1, 1, FF), lambda j, e, lo, hi: (e[j], 0, 0)),
        ],
        out_specs=pl.BlockSpec((TILE_M, FF), lambda j, e, lo, hi: (j // NE, 0)),
    )
    hid = pl.pallas_call(
        _ffn1_body,
        grid_spec=spec1,
        out_shape=jax.ShapeDtypeStruct((SEQ, FF), jnp.float32),
        compiler_params=pltpu.CompilerParams(
            dimension_semantics=("arbitrary",)),
    )(e_eff, lo, hi, xs, w1, b1)
    spec2 = pltpu.PrefetchScalarGridSpec(
        num_scalar_prefetch=3,
        grid=(NUM_ITEMS,),
        in_specs=[
            pl.BlockSpec((TILE_M, FF), lambda j, e, lo, hi: (j // NE, 0)),
            pl.BlockSpec((1, FF, DM), lambda j, e, lo, hi: (e[j], 0, 0)),
            pl.BlockSpec((1, 1, DM), lambda j, e, lo, hi: (e[j], 0, 0)),
        ],
        out_specs=pl.BlockSpec((TILE_M, DM), lambda j, e, lo, hi: (j // NE, 0)),
    )
    return pl.pallas_call(
        _ffn2_body,
        grid_spec=spec2,
        out_shape=jax.ShapeDtypeStruct((SEQ, DM), jnp.float32),
        compiler_params=pltpu.CompilerParams(
            dimension_semantics=("arbitrary",)),
    )(e_eff, lo, hi, hid, w2, b2)


def _head_body(xf_ref, wh_ref, bh_ref, out_ref):
    acc = jnp.dot(xf_ref[...].astype(jnp.bfloat16),
                  wh_ref[...].astype(jnp.bfloat16),
                  preferred_element_type=jnp.float32)
    out_ref[...] = acc + bh_ref[...]


def _head_call(xf, wh, bh):
    v = wh.shape[1]
    nv = pl.cdiv(v, VT)
    return pl.pallas_call(
        _head_body,
        grid=(nv,),
        in_specs=[
            pl.BlockSpec((SEQ, DM), lambda j: (0, 0)),
            pl.BlockSpec((DM, VT), lambda j: (0, j)),
            pl.BlockSpec((1, VT), lambda j: (0, j)),
        ],
        out_specs=pl.BlockSpec((SEQ, VT), lambda j: (0, j)),
        out_shape=jax.ShapeDtypeStruct((SEQ, v), jnp.float32),
        compiler_params=pltpu.CompilerParams(
            dimension_semantics=("arbitrary",)),
    )(xf, wh, bh)


def _ffn_items(starts):
    s = starts.reshape(NE)
    ends = jnp.concatenate([s[1:], jnp.array([SEQ], jnp.int32)])
    j = jnp.arange(NUM_ITEMS, dtype=jnp.int32)
    m = j // NE
    e = j % NE
    glo = jnp.maximum(s[e], m * TILE_M)
    ghi = jnp.minimum(ends[e], (m + 1) * TILE_M)
    valid = ghi > glo
    lo = jnp.where(valid, glo - m * TILE_M, 0).astype(jnp.int32)
    hi = jnp.where(valid, ghi - m * TILE_M, 0).astype(jnp.int32)
    marker = jnp.where(valid, j, -1)
    last_valid = lax.cummax(marker, axis=0)
    first_valid = jnp.argmax(valid).astype(jnp.int32)
    last_valid = jnp.maximum(last_valid, first_valid)
    e_eff = e[last_valid].astype(jnp.int32)
    return e_eff, lo, hi


def kernel(ids, emb, pos_emb, ln_g, ln_b, Wr, br, W1, b1, W2, b2, fg, fb,
           Wh, bh):
    b_, s_ = ids.shape
    n = b_ * s_
    ids_flat = ids.reshape(n).astype(jnp.int32)

    emb_gather = _make_sc_gather(n)
    row_gather = _make_sc_gather(n)
    row_scatter = _make_sc_scatter(n)

    x_rows = emb_gather(emb, ids_flat)
    x = x_rows + pos_emb[:s_]

    idx_layers = []
    probs_layers = []
    for i in range(ln_g.shape[0]):
        mu = jnp.mean(x, axis=-1, keepdims=True)
        var = jnp.var(x, axis=-1, keepdims=True)
        h, ex, idx, pos, starts = _router_call(
            x, mu, var,
            ln_g[i].reshape(1, DM), ln_b[i].reshape(1, DM),
            Wr[i], br[i].reshape(1, NE))
        probs = ex / jnp.sum(ex, axis=-1, keepdims=True)
        gate = jnp.max(probs, axis=-1, keepdims=True)
        e_eff, lo, hi = _ffn_items(starts)
        pos1 = pos.reshape(n)
        xs = row_scatter(h, pos1)
        ys = _ffn_call(e_eff, lo, hi, xs, W1[i],
                       b1[i].reshape(NE, 1, FF), W2[i],
                       b2[i].reshape(NE, 1, DM))
        ysg = row_gather(ys, pos1)
        disp = jax.nn.one_hot(idx.reshape(n), NE, dtype=jnp.float32)
        comb = jnp.einsum('te,etd->td', disp * gate,
                          jnp.broadcast_to(ysg[None], (NE, n, DM)))
        x = x + comb
        idx_layers.append(idx.reshape(b_, s_))
        probs_layers.append(probs.reshape(b_, s_, NE))

    mu = jnp.mean(x, axis=-1, keepdims=True)
    var = jnp.var(x, axis=-1, keepdims=True)
    xf = _final_ln_call(x, mu, var,
                        fg.reshape(1, DM), fb.reshape(1, DM))
    out2d = _head_call(xf, Wh, bh.reshape(1, -1))
    out = out2d.reshape(b_, s_, Wh.shape[1])
    return (out, jnp.stack(idx_layers), jnp.stack(probs_layers))
